# initial kernel scaffold (unmeasured)
import jax
import jax.numpy as jnp
from jax import lax
from jax.experimental import pallas as pl
from jax.experimental.pallas import tpu as pltpu

N_DEV = 32
BR = 64
D = 1024
H = 2048
B = N_DEV * BR


def kernel(x, Win0, Wout0, Win1, Wout1, Win2, Wout2):
    xb = x.astype(jnp.bfloat16)
    wi = [w.astype(jnp.bfloat16) for w in (Win0, Win1, Win2)]
    wo = [w.astype(jnp.bfloat16) for w in (Wout0, Wout1, Wout2)]

    def body(x_ref, wi0, wi1, wi2, wo0, wo1, wo2, out_ref,
             xfull, h_ref, part_ref, rs_buf,
             sA_s, sA_r, sRS_s, sRS_r, sAG_s, sAG_r):
        me = lax.axis_index("i")

        def peer(d):
            return lax.rem(me + d, N_DEV)

        rows = lambda i: (pl.ds(i * BR, BR), slice(None))

        xfull[rows(me)] = x_ref[...]

        def a_start(d, c):
            p = peer(d)
            pltpu.make_async_remote_copy(
                src_ref=x_ref,
                dst_ref=xfull.at[rows(me)],
                send_sem=sA_s.at[p],
                recv_sem=sA_r.at[me],
                device_id=(p,),
                device_id_type=pl.DeviceIdType.MESH,
            ).start()
            return c
        lax.fori_loop(1, N_DEV, a_start, 0)

        def a_wait_recv(d, c):
            p = peer(d)
            pltpu.make_async_remote_copy(
                src_ref=x_ref,
                dst_ref=xfull.at[rows(p)],
                send_sem=sA_s.at[p],
                recv_sem=sA_r.at[p],
                device_id=(p,),
                device_id_type=pl.DeviceIdType.MESH,
            ).wait_recv()
            return c
        lax.fori_loop(1, N_DEV, a_wait_recv, 0)

        def a_wait_send(d, c):
            p = peer(d)
            pltpu.make_async_remote_copy(
                src_ref=x_ref,
                dst_ref=xfull.at[rows(me)],
                send_sem=sA_s.at[p],
                recv_sem=sA_r.at[me],
                device_id=(p,),
                device_id_type=pl.DeviceIdType.MESH,
            ).wait_send()
            return c
        lax.fori_loop(1, N_DEV, a_wait_send, 0)

        for k, (win, wout) in enumerate(((wi0, wo0), (wi1, wo1), (wi2, wo2))):
            h = jnp.dot(xfull[...], win[...],
                        preferred_element_type=jnp.float32)
            h_ref[...] = jnp.maximum(h, 0.0).astype(jnp.bfloat16)
            part = jnp.dot(h_ref[...], wout[...],
                           preferred_element_type=jnp.float32)
            part_ref[...] = part.astype(jnp.bfloat16)

            rs_buf[rows(me)] = part_ref[rows(me)]

            def rs_start(d, c):
                p = peer(d)
                pltpu.make_async_remote_copy(
                    src_ref=part_ref.at[rows(p)],
                    dst_ref=rs_buf.at[rows(me)],
                    send_sem=sRS_s.at[p],
                    recv_sem=sRS_r.at[me],
                    device_id=(p,),
                    device_id_type=pl.DeviceIdType.MESH,
                ).start()
                return c
            lax.fori_loop(1, N_DEV, rs_start, 0)

            def rs_wait_recv(d, c):
                p = peer(d)
                pltpu.make_async_remote_copy(
                    src_ref=part_ref.at[rows(me)],
                    dst_ref=rs_buf.at[rows(p)],
                    send_sem=sRS_s.at[p],
                    recv_sem=sRS_r.at[p],
                    device_id=(p,),
                    device_id_type=pl.DeviceIdType.MESH,
                ).wait_recv()
                return c
            lax.fori_loop(1, N_DEV, rs_wait_recv, 0)

            def rs_wait_send(d, c):
                p = peer(d)
                pltpu.make_async_remote_copy(
                    src_ref=part_ref.at[rows(p)],
                    dst_ref=rs_buf.at[rows(me)],
                    send_sem=sRS_s.at[p],
                    recv_sem=sRS_r.at[me],
                    device_id=(p,),
                    device_id_type=pl.DeviceIdType.MESH,
                ).wait_send()
                return c
            lax.fori_loop(1, N_DEV, rs_wait_send, 0)

            red = jnp.sum(
                rs_buf[...].astype(jnp.float32).reshape(N_DEV, BR, D),
                axis=0,
            )

            tgt = xfull if k < 2 else out_ref
            tgt[rows(me)] = red.astype(jnp.bfloat16)

            def ag_start(d, c):
                p = peer(d)
                pltpu.make_async_remote_copy(
                    src_ref=tgt.at[rows(me)],
                    dst_ref=tgt.at[rows(me)],
                    send_sem=sAG_s.at[p],
                    recv_sem=sAG_r.at[me],
                    device_id=(p,),
                    device_id_type=pl.DeviceIdType.MESH,
                ).start()
                return c
            lax.fori_loop(1, N_DEV, ag_start, 0)

            def ag_wait_recv(d, c):
                p = peer(d)
                pltpu.make_async_remote_copy(
                    src_ref=tgt.at[rows(me)],
                    dst_ref=tgt.at[rows(p)],
                    send_sem=sAG_s.at[p],
                    recv_sem=sAG_r.at[p],
                    device_id=(p,),
                    device_id_type=pl.DeviceIdType.MESH,
                ).wait_recv()
                return c
            lax.fori_loop(1, N_DEV, ag_wait_recv, 0)

            def ag_wait_send(d, c):
                p = peer(d)
                pltpu.make_async_remote_copy(
                    src_ref=tgt.at[rows(me)],
                    dst_ref=tgt.at[rows(me)],
                    send_sem=sAG_s.at[p],
                    recv_sem=sAG_r.at[me],
                    device_id=(p,),
                    device_id_type=pl.DeviceIdType.MESH,
                ).wait_send()
                return c
            lax.fori_loop(1, N_DEV, ag_wait_send, 0)

    vmem = pl.BlockSpec(memory_space=pltpu.VMEM)
    out = pl.pallas_call(
        body,
        out_shape=jax.ShapeDtypeStruct((B, D), jnp.bfloat16),
        in_specs=[vmem] * 7,
        out_specs=vmem,
        scratch_shapes=[
            pltpu.VMEM((B, D), jnp.bfloat16),
            pltpu.VMEM((B, H), jnp.bfloat16),
            pltpu.VMEM((B, D), jnp.bfloat16),
            pltpu.VMEM((B, D), jnp.bfloat16),
            pltpu.SemaphoreType.DMA((N_DEV,)),
            pltpu.SemaphoreType.DMA((N_DEV,)),
            pltpu.SemaphoreType.DMA((N_DEV,)),
            pltpu.SemaphoreType.DMA((N_DEV,)),
            pltpu.SemaphoreType.DMA((N_DEV,)),
            pltpu.SemaphoreType.DMA((N_DEV,)),
        ],
    )(xb, wi[0], wi[1], wi[2], wo[0], wo[1], wo[2])
    return out.astype(jnp.float32)


# baseline (device time: 482112 ns/iter reference)
import jax
import jax.numpy as jnp
from jax import lax
from jax.experimental import pallas as pl
from jax.experimental.pallas import tpu as pltpu

N_DEV = 32
BR = 64
D = 1024
H = 2048
B = N_DEV * BR


def kernel(x, Win0, Wout0, Win1, Wout1, Win2, Wout2):
    xb = x.astype(jnp.bfloat16)
    wi = [w.astype(jnp.bfloat16) for w in (Win0, Win1, Win2)]
    wo = [w.astype(jnp.bfloat16) for w in (Wout0, Wout1, Wout2)]

    def body(x_ref, wi0, wi1, wi2, wo0, wo1, wo2, out_ref,
             xfull, h_ref, part_ref, rs_buf,
             sA_s, sA_r, sRS_s, sRS_r, sAG_s, sAG_r):
        me = lax.axis_index("i")

        def peer(d):
            return lax.rem(me + d, N_DEV)

        rows = lambda i: (pl.ds(i * BR, BR), slice(None))

        barrier_sem = pltpu.get_barrier_semaphore()

        def bar(d, c):
            pl.semaphore_signal(
                barrier_sem, inc=1,
                device_id=(peer(d),),
                device_id_type=pl.DeviceIdType.MESH,
            )
            return c
        lax.fori_loop(1, N_DEV, bar, 0)
        pl.semaphore_wait(barrier_sem, N_DEV - 1)

        xfull[rows(me)] = x_ref[...]

        def a_start(d, c):
            p = peer(d)
            pltpu.make_async_remote_copy(
                src_ref=x_ref,
                dst_ref=xfull.at[rows(me)],
                send_sem=sA_s.at[p],
                recv_sem=sA_r.at[me],
                device_id=(p,),
                device_id_type=pl.DeviceIdType.MESH,
            ).start()
            return c
        lax.fori_loop(1, N_DEV, a_start, 0)

        def a_wait_recv(d, c):
            p = peer(d)
            pltpu.make_async_remote_copy(
                src_ref=x_ref,
                dst_ref=xfull.at[rows(p)],
                send_sem=sA_s.at[p],
                recv_sem=sA_r.at[p],
                device_id=(p,),
                device_id_type=pl.DeviceIdType.MESH,
            ).wait_recv()
            return c
        lax.fori_loop(1, N_DEV, a_wait_recv, 0)

        def a_wait_send(d, c):
            p = peer(d)
            pltpu.make_async_remote_copy(
                src_ref=x_ref,
                dst_ref=xfull.at[rows(me)],
                send_sem=sA_s.at[p],
                recv_sem=sA_r.at[me],
                device_id=(p,),
                device_id_type=pl.DeviceIdType.MESH,
            ).wait_send()
            return c
        lax.fori_loop(1, N_DEV, a_wait_send, 0)

        for k, (win, wout) in enumerate(((wi0, wo0), (wi1, wo1), (wi2, wo2))):
            h = jnp.dot(xfull[...], win[...],
                        preferred_element_type=jnp.float32)
            h_ref[...] = jnp.maximum(h, 0.0).astype(jnp.bfloat16)
            part = jnp.dot(h_ref[...], wout[...],
                           preferred_element_type=jnp.float32)
            part_ref[...] = part.astype(jnp.bfloat16)

            rs_buf[rows(me)] = part_ref[rows(me)]

            def rs_start(d, c):
                p = peer(d)
                pltpu.make_async_remote_copy(
                    src_ref=part_ref.at[rows(p)],
                    dst_ref=rs_buf.at[rows(me)],
                    send_sem=sRS_s.at[p],
                    recv_sem=sRS_r.at[me],
                    device_id=(p,),
                    device_id_type=pl.DeviceIdType.MESH,
                ).start()
                return c
            lax.fori_loop(1, N_DEV, rs_start, 0)

            def rs_wait_recv(d, c):
                p = peer(d)
                pltpu.make_async_remote_copy(
                    src_ref=part_ref.at[rows(me)],
                    dst_ref=rs_buf.at[rows(p)],
                    send_sem=sRS_s.at[p],
                    recv_sem=sRS_r.at[p],
                    device_id=(p,),
                    device_id_type=pl.DeviceIdType.MESH,
                ).wait_recv()
                return c
            lax.fori_loop(1, N_DEV, rs_wait_recv, 0)

            def rs_wait_send(d, c):
                p = peer(d)
                pltpu.make_async_remote_copy(
                    src_ref=part_ref.at[rows(p)],
                    dst_ref=rs_buf.at[rows(me)],
                    send_sem=sRS_s.at[p],
                    recv_sem=sRS_r.at[me],
                    device_id=(p,),
                    device_id_type=pl.DeviceIdType.MESH,
                ).wait_send()
                return c
            lax.fori_loop(1, N_DEV, rs_wait_send, 0)

            red = jnp.sum(
                rs_buf[...].astype(jnp.float32).reshape(N_DEV, BR, D),
                axis=0,
            )

            tgt = xfull if k < 2 else out_ref
            tgt[rows(me)] = red.astype(jnp.bfloat16)

            def ag_start(d, c):
                p = peer(d)
                pltpu.make_async_remote_copy(
                    src_ref=tgt.at[rows(me)],
                    dst_ref=tgt.at[rows(me)],
                    send_sem=sAG_s.at[p],
                    recv_sem=sAG_r.at[me],
                    device_id=(p,),
                    device_id_type=pl.DeviceIdType.MESH,
                ).start()
                return c
            lax.fori_loop(1, N_DEV, ag_start, 0)

            def ag_wait_recv(d, c):
                p = peer(d)
                pltpu.make_async_remote_copy(
                    src_ref=tgt.at[rows(me)],
                    dst_ref=tgt.at[rows(p)],
                    send_sem=sAG_s.at[p],
                    recv_sem=sAG_r.at[p],
                    device_id=(p,),
                    device_id_type=pl.DeviceIdType.MESH,
                ).wait_recv()
                return c
            lax.fori_loop(1, N_DEV, ag_wait_recv, 0)

            def ag_wait_send(d, c):
                p = peer(d)
                pltpu.make_async_remote_copy(
                    src_ref=tgt.at[rows(me)],
                    dst_ref=tgt.at[rows(me)],
                    send_sem=sAG_s.at[p],
                    recv_sem=sAG_r.at[me],
                    device_id=(p,),
                    device_id_type=pl.DeviceIdType.MESH,
                ).wait_send()
                return c
            lax.fori_loop(1, N_DEV, ag_wait_send, 0)

    vmem = pl.BlockSpec(memory_space=pltpu.VMEM)
    out = pl.pallas_call(
        body,
        out_shape=jax.ShapeDtypeStruct((B, D), jnp.bfloat16),
        in_specs=[vmem] * 7,
        out_specs=vmem,
        scratch_shapes=[
            pltpu.VMEM((B, D), jnp.bfloat16),
            pltpu.VMEM((B, H), jnp.bfloat16),
            pltpu.VMEM((B, D), jnp.bfloat16),
            pltpu.VMEM((B, D), jnp.bfloat16),
            pltpu.SemaphoreType.DMA((N_DEV,)),
            pltpu.SemaphoreType.DMA((N_DEV,)),
            pltpu.SemaphoreType.DMA((N_DEV,)),
            pltpu.SemaphoreType.DMA((N_DEV,)),
            pltpu.SemaphoreType.DMA((N_DEV,)),
            pltpu.SemaphoreType.DMA((N_DEV,)),
        ],
        compiler_params=pltpu.CompilerParams(
            vmem_limit_bytes=128 * 1024 * 1024,
            collective_id=0,
        ),
    )(xb, wi[0], wi[1], wi[2], wo[0], wo[1], wo[2])
    return out.astype(jnp.float32)
